# Initial kernel scaffold; baseline (speedup 1.0000x reference)
import jax
import jax.numpy as jnp
from jax.experimental import pallas as pl


def kernel(node_feat, node_opcode, edge_index, config_feat, n_configs, batch, params):
    raise NotImplementedError("write your pallas kernel here")


# speedup vs baseline: 3.3361x; 3.3361x over previous
import functools

import jax
import jax.numpy as jnp
from jax import lax
from jax.experimental import pallas as pl
from jax.experimental.pallas import tpu as pltpu
from jax.experimental.pallas import tpu_sc as plsc

N = 50000
E = 800000
D = 64
NG = 16
N_OPS = 120

NC = 2
NS = 16
H = N // NC
TRASH = 256
ACC_ROWS = 25280
ZROWS = 320
ZCHUNKS = ACC_ROWS // ZROWS
CHUNK = 128
CPT = 391
E_PAD = NS * CPT * CHUNK
OUT_CHUNK = 1000

R = 400
GRID = N // R

_HI = jax.lax.Precision.HIGHEST


def _sc_mesh():
    return plsc.VectorSubcoreMesh(core_axis_name="c", subcore_axis_name="s")


def _seg_sum_body(tbl_hbm, ig_hbm, is_hbm, out_hbm, ig_v, is_v, il_v, rows_v,
                  z_v, acc, sem):
    c = lax.axis_index("c")
    s = lax.axis_index("s")
    base_node = c * H

    def zb(i, _):
        r = i // 4
        j = i - r * 4
        z_v[r, pl.ds(j * 16, 16)] = jnp.zeros((16,), jnp.float32)
        return 0
    lax.fori_loop(0, ZROWS * 4, zb, 0)
    for t in range(5):
        cid = s * 5 + t

        @pl.when(cid < ZCHUNKS)
        def _():
            pltpu.sync_copy(z_v, acc.at[pl.ds(cid * ZROWS, ZROWS)])
    plsc.subcore_barrier()

    def body(j, _):
        off = s * (CPT * CHUNK) + j * CHUNK
        pltpu.sync_copy(ig_hbm.at[pl.ds(off, CHUNK)], ig_v)
        pltpu.sync_copy(is_hbm.at[pl.ds(off, CHUNK)], is_v)

        def ib(i, _):
            d = is_v[pl.ds(i * 16, 16)]
            loc = d - base_node
            ok = (loc >= 0) & (loc < H)
            tr = H + (loc & (TRASH - 1))
            il_v[pl.ds(i * 16, 16)] = jnp.where(ok, loc, tr)
            return 0
        lax.fori_loop(0, CHUNK // 16, ib, 0, unroll=True)

        pltpu.async_copy(tbl_hbm.at[ig_v], rows_v, sem).wait()
        pltpu.sync_copy(rows_v, acc.at[il_v], add=True)
        return 0
    lax.fori_loop(0, CPT, body, 0)

    plsc.subcore_barrier()

    for t in range(2):
        cid = s + t * NS

        @pl.when(cid < H // OUT_CHUNK)
        def _():
            r0 = cid * OUT_CHUNK
            pltpu.sync_copy(acc.at[pl.ds(r0, OUT_CHUNK)],
                            out_hbm.at[pl.ds(base_node + r0, OUT_CHUNK)])


@jax.jit
def _seg_sum(tbl, idx_g, idx_s):
    return pl.kernel(
        _seg_sum_body,
        out_type=jax.ShapeDtypeStruct((N, D), jnp.float32),
        mesh=_sc_mesh(),
        compiler_params=pltpu.CompilerParams(use_tc_tiling_on_sc=False),
        scratch_types=[
            pltpu.VMEM((CHUNK,), jnp.int32),
            pltpu.VMEM((CHUNK,), jnp.int32),
            pltpu.VMEM((CHUNK,), jnp.int32),
            pltpu.VMEM((CHUNK, D), jnp.float32),
            pltpu.VMEM((ZROWS, D), jnp.float32),
            pltpu.VMEM_SHARED((ACC_ROWS, D), jnp.float32),
            pltpu.SemaphoreType.DMA,
        ],
    )(tbl, idx_g, idx_s)


CD = 16


def _counts_body(is1_hbm, is2_hbm, out1_hbm, out2_hbm, i1_v, i2_v, il_v,
                 ones_v, z_v, acc1, acc2, sem):
    c = lax.axis_index("c")
    s = lax.axis_index("s")
    base_node = c * H

    def zb(i, _):
        z_v[i, pl.ds(0, 16)] = jnp.zeros((16,), jnp.float32)
        ones_v[i % CHUNK, pl.ds(0, 16)] = jnp.ones((16,), jnp.float32)
        return 0
    lax.fori_loop(0, ZROWS, zb, 0)
    for t in range(5):
        cid = s * 5 + t

        @pl.when(cid < ZCHUNKS)
        def _():
            pltpu.sync_copy(z_v, acc1.at[pl.ds(cid * ZROWS, ZROWS)])
            pltpu.sync_copy(z_v, acc2.at[pl.ds(cid * ZROWS, ZROWS)])
    plsc.subcore_barrier()

    def localize(iv, i):
        d = iv[pl.ds(i * 16, 16)]
        loc = d - base_node
        ok = (loc >= 0) & (loc < H)
        tr = H + (loc & (TRASH - 1))
        return jnp.where(ok, loc, tr)

    def body(j, _):
        off = s * (CPT * CHUNK) + j * CHUNK
        pltpu.sync_copy(is1_hbm.at[pl.ds(off, CHUNK)], i1_v)
        pltpu.sync_copy(is2_hbm.at[pl.ds(off, CHUNK)], i2_v)

        def ib1(i, _):
            il_v[pl.ds(i * 16, 16)] = localize(i1_v, i)
            return 0
        lax.fori_loop(0, CHUNK // 16, ib1, 0, unroll=True)
        pltpu.sync_copy(ones_v, acc1.at[il_v], add=True)

        def ib2(i, _):
            il_v[pl.ds(i * 16, 16)] = localize(i2_v, i)
            return 0
        lax.fori_loop(0, CHUNK // 16, ib2, 0, unroll=True)
        pltpu.sync_copy(ones_v, acc2.at[il_v], add=True)
        return 0
    lax.fori_loop(0, CPT, body, 0)

    plsc.subcore_barrier()

    for t in range(2):
        cid = s + t * NS

        @pl.when(cid < H // OUT_CHUNK)
        def _():
            r0 = cid * OUT_CHUNK
            pltpu.sync_copy(acc1.at[pl.ds(r0, OUT_CHUNK)],
                            out1_hbm.at[pl.ds(base_node + r0, OUT_CHUNK)])
            pltpu.sync_copy(acc2.at[pl.ds(r0, OUT_CHUNK)],
                            out2_hbm.at[pl.ds(base_node + r0, OUT_CHUNK)])


@jax.jit
def _counts(idx_s1, idx_s2):
    return pl.kernel(
        _counts_body,
        out_type=[jax.ShapeDtypeStruct((N, CD), jnp.float32),
                  jax.ShapeDtypeStruct((N, CD), jnp.float32)],
        mesh=_sc_mesh(),
        compiler_params=pltpu.CompilerParams(use_tc_tiling_on_sc=False),
        scratch_types=[
            pltpu.VMEM((CHUNK,), jnp.int32),
            pltpu.VMEM((CHUNK,), jnp.int32),
            pltpu.VMEM((CHUNK,), jnp.int32),
            pltpu.VMEM((CHUNK, CD), jnp.float32),
            pltpu.VMEM((ZROWS, CD), jnp.float32),
            pltpu.VMEM_SHARED((ACC_ROWS, CD), jnp.float32),
            pltpu.VMEM_SHARED((ACC_ROWS, CD), jnp.float32),
            pltpu.SemaphoreType.DMA,
        ],
    )(idx_s1, idx_s2)


def _enc_body(nf_ref, opc_ref, ope_ref, she_ref, w_ref, b_ref,
              yf_ref, yb_ref, yr_ref):
    nf = nf_ref[...]
    opc = opc_ref[0, 0, :]
    oh_op = (opc[:, None] == lax.broadcasted_iota(jnp.int32, (1, N_OPS), 1)
             ).astype(jnp.float32)
    xop = jnp.dot(oh_op, ope_ref[...], preferred_element_type=jnp.float32,
                  precision=_HI)
    sidx = nf[:, 139].astype(jnp.int32)
    oh_s = (sidx[:, None] == lax.broadcasted_iota(jnp.int32, (1, 8), 1)
            ).astype(jnp.float32)
    xsh = jnp.dot(oh_s, she_ref[...], preferred_element_type=jnp.float32,
                  precision=_HI)
    x = jnp.concatenate([nf[:, :139], xop, xsh], axis=1)
    y = jnp.dot(x, w_ref[...], preferred_element_type=jnp.float32,
                precision=_HI) + b_ref[...]
    yf_ref[...] = y[:, :D]
    yb_ref[...] = y[:, D:2 * D]
    yr_ref[...] = y[:, 2 * D:]


def _encode_project(node_feat, opcode3, op_emb, shape_emb, wcat, bias):
    return pl.pallas_call(
        _enc_body,
        grid=(GRID,),
        in_specs=[
            pl.BlockSpec((R, 140), lambda i: (i, 0)),
            pl.BlockSpec((1, 1, R), lambda i: (i, 0, 0)),
            pl.BlockSpec((N_OPS, 8), lambda i: (0, 0)),
            pl.BlockSpec((8, 4), lambda i: (0, 0)),
            pl.BlockSpec((151, 3 * D), lambda i: (0, 0)),
            pl.BlockSpec((1, 3 * D), lambda i: (0, 0)),
        ],
        out_specs=[pl.BlockSpec((R, D), lambda i: (i, 0))] * 3,
        out_shape=[jax.ShapeDtypeStruct((N, D), jnp.float32)] * 3,
    )(node_feat, opcode3, op_emb, shape_emb, wcat, bias)


def _comb_body(af_ref, ab_ref, cf_ref, cb_ref, yr_ref, w_ref, b_ref,
               yf_o, yb_o, yr_o):
    rf = 1.0 / jnp.maximum(cf_ref[:, :1], 1.0)
    rb = 1.0 / jnp.maximum(cb_ref[:, :1], 1.0)
    x = jnp.maximum(af_ref[...] * rf + ab_ref[...] * rb + yr_ref[...], 0.0)
    y = jnp.dot(x, w_ref[...], preferred_element_type=jnp.float32,
                precision=_HI) + b_ref[...]
    yf_o[...] = y[:, :D]
    yb_o[...] = y[:, D:2 * D]
    yr_o[...] = y[:, 2 * D:]


def _combine_project(aggf, aggb, cntf, cntb, yr, wcat, bias):
    return pl.pallas_call(
        _comb_body,
        grid=(GRID,),
        in_specs=[
            pl.BlockSpec((R, D), lambda i: (i, 0)),
            pl.BlockSpec((R, D), lambda i: (i, 0)),
            pl.BlockSpec((R, CD), lambda i: (i, 0)),
            pl.BlockSpec((R, CD), lambda i: (i, 0)),
            pl.BlockSpec((R, D), lambda i: (i, 0)),
            pl.BlockSpec((D, 3 * D), lambda i: (0, 0)),
            pl.BlockSpec((1, 3 * D), lambda i: (0, 0)),
        ],
        out_specs=[pl.BlockSpec((R, D), lambda i: (i, 0))] * 3,
        out_shape=[jax.ShapeDtypeStruct((N, D), jnp.float32)] * 3,
    )(aggf, aggb, cntf, cntb, yr, wcat, bias)


def _head_body(af_ref, ab_ref, cf_ref, cb_ref, yr_ref, bat_ref, cfg_ref,
               w1_ref, b1_ref, w2_ref, b2_ref, out_ref,
               ssum, smax, scnt):
    i = pl.program_id(0)

    @pl.when(i == 0)
    def _():
        ssum[...] = jnp.zeros((NG, D), jnp.float32)
        smax[...] = jnp.zeros((NG, D), jnp.float32)
        scnt[...] = jnp.zeros((NG, 1), jnp.float32)

    rf = 1.0 / jnp.maximum(cf_ref[:, :1], 1.0)
    rb = 1.0 / jnp.maximum(cb_ref[:, :1], 1.0)
    x = jnp.maximum(af_ref[...] * rf + ab_ref[...] * rb + yr_ref[...], 0.0)
    b = bat_ref[0, 0, :]
    oh = (b[:, None] == lax.broadcasted_iota(jnp.int32, (1, NG), 1)
          ).astype(jnp.float32)
    ssum[...] += lax.dot_general(oh, x, (((0,), (0,)), ((), ())),
                                 preferred_element_type=jnp.float32,
                                 precision=_HI)
    scnt[...] += lax.dot_general(oh, jnp.ones((R, 1), jnp.float32),
                                 (((0,), (0,)), ((), ())),
                                 preferred_element_type=jnp.float32,
                                 precision=_HI)
    for g in range(NG):
        m = jnp.where(b[:, None] == g, x, 0.0)
        smax[g:g + 1, :] = jnp.maximum(smax[g:g + 1, :],
                                       jnp.max(m, axis=0, keepdims=True))

    @pl.when(i == GRID - 1)
    def _():
        mean = ssum[...] / jnp.maximum(scnt[...], 1.0)
        gf = jnp.concatenate([mean, smax[...]], axis=1)
        hc = jnp.concatenate([gf, cfg_ref[...]], axis=1)
        h = jnp.maximum(jnp.dot(hc, w1_ref[...],
                                preferred_element_type=jnp.float32,
                                precision=_HI) + b1_ref[...], 0.0)
        out_ref[...] = jnp.dot(h, w2_ref[...],
                               preferred_element_type=jnp.float32,
                               precision=_HI) + b2_ref[...]


def _pool_head(aggf, aggb, cntf, cntb, yr, batch3, config_feat, w1, b1, w2, b2):
    return pl.pallas_call(
        _head_body,
        grid=(GRID,),
        in_specs=[
            pl.BlockSpec((R, D), lambda i: (i, 0)),
            pl.BlockSpec((R, D), lambda i: (i, 0)),
            pl.BlockSpec((R, CD), lambda i: (i, 0)),
            pl.BlockSpec((R, CD), lambda i: (i, 0)),
            pl.BlockSpec((R, D), lambda i: (i, 0)),
            pl.BlockSpec((1, 1, R), lambda i: (i, 0, 0)),
            pl.BlockSpec((NG, 24), lambda i: (0, 0)),
            pl.BlockSpec((2 * D + 24, D), lambda i: (0, 0)),
            pl.BlockSpec((1, D), lambda i: (0, 0)),
            pl.BlockSpec((D, 1), lambda i: (0, 0)),
            pl.BlockSpec((1, 1), lambda i: (0, 0)),
        ],
        out_specs=pl.BlockSpec((NG, 1), lambda i: (0, 0)),
        out_shape=jax.ShapeDtypeStruct((NG, 1), jnp.float32),
        scratch_shapes=[
            pltpu.VMEM((NG, D), jnp.float32),
            pltpu.VMEM((NG, D), jnp.float32),
            pltpu.VMEM((NG, 1), jnp.float32),
        ],
    )(aggf, aggb, cntf, cntb, yr, batch3, config_feat, w1, b1, w2, b2)


def kernel(node_feat, node_opcode, edge_index, config_feat, n_configs, batch,
           params):
    src = edge_index[:, 0]
    dst = edge_index[:, 1]
    pad0 = jnp.zeros((E_PAD - E,), jnp.int32)
    padm = jnp.full((E_PAD - E,), -1, jnp.int32)
    src_g = jnp.concatenate([src, pad0])
    dst_g = jnp.concatenate([dst, pad0])
    src_s = jnp.concatenate([src, padm])
    dst_s = jnp.concatenate([dst, padm])

    opcode3 = node_opcode.reshape(GRID, 1, R)
    batch3 = batch.reshape(GRID, 1, R)

    wcats, biases = [], []
    for lyr in params['layers']:
        wcats.append(jnp.concatenate(
            [lyr['Wn_f'], lyr['Wn_b'], lyr['Wr_f'] + lyr['Wr_b']], axis=1))
        biases.append(jnp.concatenate(
            [jnp.zeros((2 * D,), jnp.float32), lyr['b_f'] + lyr['b_b']]
        ).reshape(1, 3 * D))

    cntf, cntb = _counts(dst_s, src_s)

    yf, yb, yr = _encode_project(node_feat, opcode3, params['op_emb'],
                                 params['shape_emb'], wcats[0], biases[0])
    for l in (1, 2):
        aggf = _seg_sum(yf, src_g, dst_s)
        aggb = _seg_sum(yb, dst_g, src_s)
        yf, yb, yr = _combine_project(aggf, aggb, cntf, cntb, yr,
                                      wcats[l], biases[l])
    aggf = _seg_sum(yf, src_g, dst_s)
    aggb = _seg_sum(yb, dst_g, src_s)

    out = _pool_head(aggf, aggb, cntf, cntb, yr, batch3, config_feat,
                     params['W1'], params['b1'].reshape(1, D),
                     params['W2'], params['b2'].reshape(1, 1))
    return out[:, 0]


# speedup vs baseline: 6.0376x; 1.8098x over previous
import functools

import jax
import jax.numpy as jnp
from jax import lax
from jax.experimental import pallas as pl
from jax.experimental.pallas import tpu as pltpu
from jax.experimental.pallas import tpu_sc as plsc

N = 50000
E = 800000
D = 64
NG = 16
N_OPS = 120

NC = 2
NS = 16
H = N // NC
TRASH = 128
ACC_ROWS = 25152
ZROWS = 64
ZCHUNKS = ACC_ROWS // ZROWS
DH = 32
CHUNK = 128
G = 8
CPT = 392
E_PAD = NS * CPT * CHUNK
NGROUP = CPT // G
OUT_CHUNK = 1000

R = 400
GRID = N // R

_HI = jax.lax.Precision.HIGHEST


def _sc_mesh():
    return plsc.VectorSubcoreMesh(core_axis_name="c", subcore_axis_name="s")


def _seg_sum_body(tbl_hbm, ig_hbm, is_hbm, out_hbm, ig_v, is_v, il_v, rows_v,
                  z_v, acc, sem_i, sem_g, sem_s):
    c = lax.axis_index("c")
    s = lax.axis_index("s")
    base_node = c * H

    zw = DH // 16
    def zb(i, _):
        r = i // zw
        j = i - r * zw
        z_v[r, pl.ds(j * 16, 16)] = jnp.zeros((16,), jnp.float32)
        return 0
    lax.fori_loop(0, ZROWS * zw, zb, 0)
    def zc(t, _):
        cid = s * (ZCHUNKS // NS + 1) + t

        @pl.when(cid < ZCHUNKS)
        def _():
            pltpu.sync_copy(z_v, acc.at[pl.ds(cid * ZROWS, ZROWS)])
        return 0
    lax.fori_loop(0, ZCHUNKS // NS + 1, zc, 0)
    plsc.subcore_barrier()

    def group(g, _):
        base = s * (CPT * CHUNK) + g * (G * CHUNK)
        d_i = []
        for b in range(G):
            d_i.append(pltpu.async_copy(
                ig_hbm.at[pl.ds(base + b * CHUNK, CHUNK)], ig_v.at[b], sem_i))
            d_i.append(pltpu.async_copy(
                is_hbm.at[pl.ds(base + b * CHUNK, CHUNK)], is_v.at[b], sem_i))
        for d in d_i:
            d.wait()
        d_g = []
        for b in range(G):
            def ib(i, _):
                dd = is_v[b, pl.ds(i * 16, 16)]
                loc = dd - base_node
                ok = (loc >= 0) & (loc < H)
                tr = H + (loc & (TRASH - 1))
                il_v[b, pl.ds(i * 16, 16)] = jnp.where(ok, loc, tr)
                return 0
            lax.fori_loop(0, CHUNK // 16, ib, 0, unroll=True)
            d_g.append(pltpu.async_copy(tbl_hbm.at[ig_v.at[b]], rows_v.at[b],
                                        sem_g))
        d_s = []
        for b in range(G):
            d_g[b].wait()
            d_s.append(pltpu.async_copy(rows_v.at[b], acc.at[il_v.at[b]],
                                        sem_s, add=True))
        for d in d_s:
            d.wait()
        return 0
    lax.fori_loop(0, NGROUP, group, 0)

    plsc.subcore_barrier()

    for t in range(2):
        cid = s + t * NS

        @pl.when(cid < H // OUT_CHUNK)
        def _():
            r0 = cid * OUT_CHUNK
            pltpu.sync_copy(acc.at[pl.ds(r0, OUT_CHUNK)],
                            out_hbm.at[pl.ds(base_node + r0, OUT_CHUNK)])


@jax.jit
def _seg_sum(tbl, idx_g, idx_s):
    return pl.kernel(
        _seg_sum_body,
        out_type=jax.ShapeDtypeStruct((N, DH), jnp.float32),
        mesh=_sc_mesh(),
        compiler_params=pltpu.CompilerParams(use_tc_tiling_on_sc=False),
        scratch_types=[
            pltpu.VMEM((G, CHUNK), jnp.int32),
            pltpu.VMEM((G, CHUNK), jnp.int32),
            pltpu.VMEM((G, CHUNK), jnp.int32),
            pltpu.VMEM((G, CHUNK, DH), jnp.float32),
            pltpu.VMEM((ZROWS, DH), jnp.float32),
            pltpu.VMEM_SHARED((ACC_ROWS, DH), jnp.float32),
            pltpu.SemaphoreType.DMA,
            pltpu.SemaphoreType.DMA,
            pltpu.SemaphoreType.DMA,
        ],
    )(tbl, idx_g, idx_s)


CD = 16


def _counts_body(is1_hbm, is2_hbm, out1_hbm, out2_hbm, i1_v, i2_v, il1_v,
                 il2_v, ones_v, z_v, acc1, acc2, sem_i, sem_s):
    c = lax.axis_index("c")
    s = lax.axis_index("s")
    base_node = c * H

    def zb(i, _):
        ones_v[i, pl.ds(0, 16)] = jnp.ones((16,), jnp.float32)

        @pl.when(i < ZROWS)
        def _():
            z_v[i, pl.ds(0, 16)] = jnp.zeros((16,), jnp.float32)
        return 0
    lax.fori_loop(0, CHUNK, zb, 0)
    def zc(t, _):
        cid = s * (ZCHUNKS // NS + 1) + t

        @pl.when(cid < ZCHUNKS)
        def _():
            pltpu.sync_copy(z_v, acc1.at[pl.ds(cid * ZROWS, ZROWS)])
            pltpu.sync_copy(z_v, acc2.at[pl.ds(cid * ZROWS, ZROWS)])
        return 0
    lax.fori_loop(0, ZCHUNKS // NS + 1, zc, 0)
    plsc.subcore_barrier()

    def group(g, _):
        base = s * (CPT * CHUNK) + g * (G * CHUNK)
        d_i = []
        for b in range(G):
            d_i.append(pltpu.async_copy(
                is1_hbm.at[pl.ds(base + b * CHUNK, CHUNK)], i1_v.at[b], sem_i))
            d_i.append(pltpu.async_copy(
                is2_hbm.at[pl.ds(base + b * CHUNK, CHUNK)], i2_v.at[b], sem_i))
        for d in d_i:
            d.wait()
        d_s = []
        for b in range(G):
            def ib(i, _):
                for iv, ol in ((i1_v, il1_v), (i2_v, il2_v)):
                    dd = iv[b, pl.ds(i * 16, 16)]
                    loc = dd - base_node
                    ok = (loc >= 0) & (loc < H)
                    tr = H + (loc & (TRASH - 1))
                    ol[b, pl.ds(i * 16, 16)] = jnp.where(ok, loc, tr)
                return 0
            lax.fori_loop(0, CHUNK // 16, ib, 0, unroll=True)
            d_s.append(pltpu.async_copy(ones_v, acc1.at[il1_v.at[b]],
                                        sem_s, add=True))
            d_s.append(pltpu.async_copy(ones_v, acc2.at[il2_v.at[b]],
                                        sem_s, add=True))
        for d in d_s:
            d.wait()
        return 0
    lax.fori_loop(0, NGROUP, group, 0)

    plsc.subcore_barrier()

    for t in range(2):
        cid = s + t * NS

        @pl.when(cid < H // OUT_CHUNK)
        def _():
            r0 = cid * OUT_CHUNK
            pltpu.sync_copy(acc1.at[pl.ds(r0, OUT_CHUNK)],
                            out1_hbm.at[pl.ds(base_node + r0, OUT_CHUNK)])
            pltpu.sync_copy(acc2.at[pl.ds(r0, OUT_CHUNK)],
                            out2_hbm.at[pl.ds(base_node + r0, OUT_CHUNK)])


@jax.jit
def _counts(idx_s1, idx_s2):
    return pl.kernel(
        _counts_body,
        out_type=[jax.ShapeDtypeStruct((N, CD), jnp.float32),
                  jax.ShapeDtypeStruct((N, CD), jnp.float32)],
        mesh=_sc_mesh(),
        compiler_params=pltpu.CompilerParams(use_tc_tiling_on_sc=False),
        scratch_types=[
            pltpu.VMEM((G, CHUNK), jnp.int32),
            pltpu.VMEM((G, CHUNK), jnp.int32),
            pltpu.VMEM((G, CHUNK), jnp.int32),
            pltpu.VMEM((G, CHUNK), jnp.int32),
            pltpu.VMEM((CHUNK, CD), jnp.float32),
            pltpu.VMEM((ZROWS, CD), jnp.float32),
            pltpu.VMEM_SHARED((ACC_ROWS, CD), jnp.float32),
            pltpu.VMEM_SHARED((ACC_ROWS, CD), jnp.float32),
            pltpu.SemaphoreType.DMA,
            pltpu.SemaphoreType.DMA,
        ],
    )(idx_s1, idx_s2)


def _split_out(y, refs):
    yfl, yfh, ybl, ybh, yr = refs
    yfl[...] = y[:, 0 * DH:1 * DH]
    yfh[...] = y[:, 1 * DH:2 * DH]
    ybl[...] = y[:, 2 * DH:3 * DH]
    ybh[...] = y[:, 3 * DH:4 * DH]
    yr[...] = y[:, 4 * DH:]


def _enc_body(nf_ref, opc_ref, ope_ref, she_ref, w_ref, b_ref,
              *out_refs):
    nf = nf_ref[...]
    opc = opc_ref[0, 0, :]
    oh_op = (opc[:, None] == lax.broadcasted_iota(jnp.int32, (1, N_OPS), 1)
             ).astype(jnp.float32)
    xop = jnp.dot(oh_op, ope_ref[...], preferred_element_type=jnp.float32,
                  precision=_HI)
    sidx = nf[:, 139].astype(jnp.int32)
    oh_s = (sidx[:, None] == lax.broadcasted_iota(jnp.int32, (1, 8), 1)
            ).astype(jnp.float32)
    xsh = jnp.dot(oh_s, she_ref[...], preferred_element_type=jnp.float32,
                  precision=_HI)
    x = jnp.concatenate([nf[:, :139], xop, xsh], axis=1)
    y = jnp.dot(x, w_ref[...], preferred_element_type=jnp.float32,
                precision=_HI) + b_ref[...]
    _split_out(y, out_refs)


_YSPECS = ([pl.BlockSpec((R, DH), lambda i: (i, 0))] * 4
           + [pl.BlockSpec((R, D), lambda i: (i, 0))])
_YSHAPES = ([jax.ShapeDtypeStruct((N, DH), jnp.float32)] * 4
            + [jax.ShapeDtypeStruct((N, D), jnp.float32)])


def _encode_project(node_feat, opcode3, op_emb, shape_emb, wcat, bias):
    return pl.pallas_call(
        _enc_body,
        grid=(GRID,),
        in_specs=[
            pl.BlockSpec((R, 140), lambda i: (i, 0)),
            pl.BlockSpec((1, 1, R), lambda i: (i, 0, 0)),
            pl.BlockSpec((N_OPS, 8), lambda i: (0, 0)),
            pl.BlockSpec((8, 4), lambda i: (0, 0)),
            pl.BlockSpec((151, 3 * D), lambda i: (0, 0)),
            pl.BlockSpec((1, 3 * D), lambda i: (0, 0)),
        ],
        out_specs=list(_YSPECS),
        out_shape=list(_YSHAPES),
    )(node_feat, opcode3, op_emb, shape_emb, wcat, bias)


def _relu_x(afl, afh, abl, abh, cf_ref, cb_ref, yr_ref):
    rf = 1.0 / jnp.maximum(cf_ref[:, :1], 1.0)
    rb = 1.0 / jnp.maximum(cb_ref[:, :1], 1.0)
    af = jnp.concatenate([afl[...], afh[...]], axis=1)
    ab = jnp.concatenate([abl[...], abh[...]], axis=1)
    return jnp.maximum(af * rf + ab * rb + yr_ref[...], 0.0)


def _comb_body(afl, afh, abl, abh, cf_ref, cb_ref, yr_ref, w_ref, b_ref,
               *out_refs):
    x = _relu_x(afl, afh, abl, abh, cf_ref, cb_ref, yr_ref)
    y = jnp.dot(x, w_ref[...], preferred_element_type=jnp.float32,
                precision=_HI) + b_ref[...]
    _split_out(y, out_refs)


_AGGSPECS = ([pl.BlockSpec((R, DH), lambda i: (i, 0))] * 4
             + [pl.BlockSpec((R, CD), lambda i: (i, 0))] * 2
             + [pl.BlockSpec((R, D), lambda i: (i, 0))])


def _combine_project(aggs, cntf, cntb, yr, wcat, bias):
    return pl.pallas_call(
        _comb_body,
        grid=(GRID,),
        in_specs=list(_AGGSPECS) + [
            pl.BlockSpec((D, 3 * D), lambda i: (0, 0)),
            pl.BlockSpec((1, 3 * D), lambda i: (0, 0)),
        ],
        out_specs=list(_YSPECS),
        out_shape=list(_YSHAPES),
    )(*aggs, cntf, cntb, yr, wcat, bias)


def _head_body(afl, afh, abl, abh, cf_ref, cb_ref, yr_ref, bat_ref, cfg_ref,
               w1_ref, b1_ref, w2_ref, b2_ref, out_ref,
               ssum, smax, scnt):
    i = pl.program_id(0)

    @pl.when(i == 0)
    def _():
        ssum[...] = jnp.zeros((NG, D), jnp.float32)
        smax[...] = jnp.zeros((NG, D), jnp.float32)
        scnt[...] = jnp.zeros((NG, 1), jnp.float32)

    x = _relu_x(afl, afh, abl, abh, cf_ref, cb_ref, yr_ref)
    b = bat_ref[0, 0, :]
    oh = (b[:, None] == lax.broadcasted_iota(jnp.int32, (1, NG), 1)
          ).astype(jnp.float32)
    ssum[...] += lax.dot_general(oh, x, (((0,), (0,)), ((), ())),
                                 preferred_element_type=jnp.float32,
                                 precision=_HI)
    scnt[...] += lax.dot_general(oh, jnp.ones((R, 1), jnp.float32),
                                 (((0,), (0,)), ((), ())),
                                 preferred_element_type=jnp.float32,
                                 precision=_HI)
    for g in range(NG):
        m = jnp.where(b[:, None] == g, x, 0.0)
        smax[g:g + 1, :] = jnp.maximum(smax[g:g + 1, :],
                                       jnp.max(m, axis=0, keepdims=True))

    @pl.when(i == GRID - 1)
    def _():
        mean = ssum[...] / jnp.maximum(scnt[...], 1.0)
        gf = jnp.concatenate([mean, smax[...]], axis=1)
        hc = jnp.concatenate([gf, cfg_ref[...]], axis=1)
        h = jnp.maximum(jnp.dot(hc, w1_ref[...],
                                preferred_element_type=jnp.float32,
                                precision=_HI) + b1_ref[...], 0.0)
        out_ref[...] = jnp.dot(h, w2_ref[...],
                               preferred_element_type=jnp.float32,
                               precision=_HI) + b2_ref[...]


def _pool_head(aggs, cntf, cntb, yr, batch3, config_feat, w1, b1, w2, b2):
    return pl.pallas_call(
        _head_body,
        grid=(GRID,),
        in_specs=list(_AGGSPECS) + [
            pl.BlockSpec((1, 1, R), lambda i: (i, 0, 0)),
            pl.BlockSpec((NG, 24), lambda i: (0, 0)),
            pl.BlockSpec((2 * D + 24, D), lambda i: (0, 0)),
            pl.BlockSpec((1, D), lambda i: (0, 0)),
            pl.BlockSpec((D, 1), lambda i: (0, 0)),
            pl.BlockSpec((1, 1), lambda i: (0, 0)),
        ],
        out_specs=pl.BlockSpec((NG, 1), lambda i: (0, 0)),
        out_shape=jax.ShapeDtypeStruct((NG, 1), jnp.float32),
        scratch_shapes=[
            pltpu.VMEM((NG, D), jnp.float32),
            pltpu.VMEM((NG, D), jnp.float32),
            pltpu.VMEM((NG, 1), jnp.float32),
        ],
    )(*aggs, cntf, cntb, yr, batch3, config_feat, w1, b1, w2, b2)


def kernel(node_feat, node_opcode, edge_index, config_feat, n_configs, batch,
           params):
    src = edge_index[:, 0]
    dst = edge_index[:, 1]
    pad0 = jnp.zeros((E_PAD - E,), jnp.int32)
    padm = jnp.full((E_PAD - E,), -1, jnp.int32)
    src_g = jnp.concatenate([src, pad0])
    dst_g = jnp.concatenate([dst, pad0])
    src_s = jnp.concatenate([src, padm])
    dst_s = jnp.concatenate([dst, padm])

    opcode3 = node_opcode.reshape(GRID, 1, R)
    batch3 = batch.reshape(GRID, 1, R)

    wcats, biases = [], []
    for lyr in params['layers']:
        wcats.append(jnp.concatenate(
            [lyr['Wn_f'], lyr['Wn_b'], lyr['Wr_f'] + lyr['Wr_b']], axis=1))
        biases.append(jnp.concatenate(
            [jnp.zeros((2 * D,), jnp.float32), lyr['b_f'] + lyr['b_b']]
        ).reshape(1, 3 * D))

    cntf, cntb = _counts(dst_s, src_s)

    def agg4(ys):
        yfl, yfh, ybl, ybh = ys
        return (_seg_sum(yfl, src_g, dst_s), _seg_sum(yfh, src_g, dst_s),
                _seg_sum(ybl, dst_g, src_s), _seg_sum(ybh, dst_g, src_s))

    *ys, yr = _encode_project(node_feat, opcode3, params['op_emb'],
                              params['shape_emb'], wcats[0], biases[0])
    for l in (1, 2):
        aggs = agg4(ys)
        *ys, yr = _combine_project(aggs, cntf, cntb, yr, wcats[l], biases[l])
    aggs = agg4(ys)

    out = _pool_head(aggs, cntf, cntb, yr, batch3, config_feat,
                     params['W1'], params['b1'].reshape(1, D),
                     params['W2'], params['b2'].reshape(1, 1))
    return out[:, 0]
